# baseline (speedup 1.0000x reference)
import functools

import jax
import jax.numpy as jnp
from jax import lax
from jax.experimental import pallas as pl
from jax.experimental.pallas import tpu as pltpu
from jax.experimental.pallas import tpu_sc as plsc

C = 16384
A = 200
NC = 2
NS = 16
NW = NC * NS
R = C // NW
CR = 128
NCH = R // CR
NFULL = A // 16
SK = 17

_mesh = plsc.VectorSubcoreMesh(core_axis_name="c", subcore_axis_name="s")


@functools.partial(
    pl.kernel,
    mesh=_mesh,
    out_type=jax.ShapeDtypeStruct((C,), jnp.float32),
    compiler_params=pltpu.CompilerParams(needs_layout_passes=False, skip_device_barrier=True),
    scratch_types=[
        pltpu.VMEM((CR, A), jnp.int32),
        pltpu.VMEM((CR, A), jnp.int32),
        pltpu.VMEM((R,), jnp.float32),
        pltpu.VMEM((R,), jnp.float32),
        pltpu.VMEM((CR * SK,), jnp.int32),
        pltpu.VMEM((CR * SK,), jnp.int32),
        pltpu.VMEM((CR * SK,), jnp.int32),
        pltpu.VMEM((128,), jnp.float32),
        pltpu.VMEM((16,), jnp.float32),
        pltpu.SemaphoreType.DMA,
        pltpu.SemaphoreType.DMA,
    ],
)
def _sc_energy_adder(species_hbm, energies_hbm, table_hbm, icpt_hbm,
                     out_hbm, sp0, sp1, en_v, out_v, psS, ps1, ps3,
                     tab_v, icpt_s, sem0, sem1):
    wid = lax.axis_index("s") * NC + lax.axis_index("c")
    row0 = wid * R
    bufs = (sp0, sp1)
    sems = (sem0, sem1)

    copies = {0: pltpu.async_copy(
        species_hbm.at[pl.ds(row0, CR)], sp0, sem0)}
    pltpu.sync_copy(table_hbm, tab_v.at[pl.ds(0, 4)])
    pltpu.sync_copy(icpt_hbm, icpt_s)
    pltpu.sync_copy(energies_hbm.at[pl.ds(row0, R)], en_v)

    icpt = icpt_s[...]
    lane = lax.iota(jnp.int32, 16)
    hi_mask = lane >= 8
    tvec = tab_v[pl.ds(0, 16)]
    t0, t1, t2, t3 = tvec[0], tvec[1], tvec[2], tvec[3]
    k1 = t1 - t0
    k2 = t2 + t0 - 2.0 * t1
    k3 = t3 - t2 - t1 + t0
    base = t0 * jnp.float32(A)
    zero_i = jnp.zeros((16,), jnp.int32)

    def quad_body(sp, q, _):
        for rr in range(4):
            row_body(sp, q * 4 + rr, None)
        return _

    def row_body(sp, r, _):
        sS0 = sS1 = s10 = s11 = s30 = s31 = zero_i
        for j in range(NFULL):
            s = sp[r, pl.ds(j * 16, 16)]
            t = s >> 1
            u = s & t
            if j % 2 == 0:
                sS0 = sS0 + s
                s10 = s10 + t
                s30 = s30 + u
            else:
                sS1 = sS1 + s
                s11 = s11 + t
                s31 = s31 + u
        s = sp[r, pl.ds(NFULL * 16 - 8, 16)]
        s = jnp.where(hi_mask, s, 0)
        t = s >> 1
        u = s & t
        psS[pl.ds(r * SK, 16)] = sS0 + sS1 + s
        ps1[pl.ds(r * SK, 16)] = s10 + s11 + t
        ps3[pl.ds(r * SK, 16)] = s30 + s31 + u
        return _

    def group_body(out_off, g, _):
        col0 = (g * 16 + lane) * SK
        accS = acc1 = acc3 = zero_i
        for j in range(0, 16, 2):
            accS = accS + plsc.load_gather(psS, [col0 + j])
            acc1 = acc1 + plsc.load_gather(ps1, [col0 + j])
            acc3 = acc3 + plsc.load_gather(ps3, [col0 + j])
            accS = accS + plsc.load_gather(psS, [col0 + (j + 1)])
            acc1 = acc1 + plsc.load_gather(ps1, [col0 + (j + 1)])
            acc3 = acc3 + plsc.load_gather(ps3, [col0 + (j + 1)])
        sae = (base
               + k1 * accS.astype(jnp.float32)
               + k2 * acc1.astype(jnp.float32)
               + k3 * acc3.astype(jnp.float32))
        sl = pl.ds(out_off + g * 16, 16)
        out_v[sl] = sae + en_v[sl] + icpt
        return _

    for c in range(NCH):
        if c + 1 < NCH:
            copies[c + 1] = pltpu.async_copy(
                species_hbm.at[pl.ds(row0 + (c + 1) * CR, CR)],
                bufs[(c + 1) % 2], sems[(c + 1) % 2])
        copies[c].wait()
        sp = bufs[c % 2]
        lax.fori_loop(0, CR // 4, functools.partial(quad_body, sp), None)
        lax.fori_loop(0, CR // 16,
                      functools.partial(group_body, c * CR), None)

    pltpu.sync_copy(out_v, out_hbm.at[pl.ds(row0, R)])


def kernel(species, energies, self_energies, intercept):
    sae = _sc_energy_adder(
        species,
        energies,
        self_energies,
        jnp.broadcast_to(intercept, (16,)),
    )
    return (species, sae)


# speedup vs baseline: 1.0296x; 1.0296x over previous
import functools

import jax
import jax.numpy as jnp
from jax import lax
from jax.experimental import pallas as pl
from jax.experimental.pallas import tpu as pltpu
from jax.experimental.pallas import tpu_sc as plsc

C = 16384
A = 200
NC = 2
NS = 16
NW = NC * NS
R = C // NW
CR = 128
NCH = R // CR
NFULL = A // 16
SK = 17

_mesh = plsc.VectorSubcoreMesh(core_axis_name="c", subcore_axis_name="s")


@functools.partial(
    pl.kernel,
    mesh=_mesh,
    out_type=jax.ShapeDtypeStruct((C,), jnp.float32),
    compiler_params=pltpu.CompilerParams(needs_layout_passes=False),
    scratch_types=[
        pltpu.VMEM((CR, A), jnp.int32),
        pltpu.VMEM((CR, A), jnp.int32),
        pltpu.VMEM((R,), jnp.float32),
        pltpu.VMEM((R,), jnp.float32),
        pltpu.VMEM((CR * SK,), jnp.int32),
        pltpu.VMEM((CR * SK,), jnp.int32),
        pltpu.VMEM((CR * SK,), jnp.int32),
        pltpu.VMEM((128,), jnp.float32),
        pltpu.VMEM((16,), jnp.float32),
        pltpu.SemaphoreType.DMA,
        pltpu.SemaphoreType.DMA,
    ],
)
def _sc_energy_adder(species_hbm, energies_hbm, table_hbm, icpt_hbm,
                     out_hbm, sp0, sp1, en_v, out_v, psS, ps1, ps3,
                     tab_v, icpt_s, sem0, sem1):
    wid = lax.axis_index("s") * NC + lax.axis_index("c")
    row0 = wid * R
    bufs = (sp0, sp1)
    sems = (sem0, sem1)

    copies = {0: pltpu.async_copy(
        species_hbm.at[pl.ds(row0, CR)], sp0, sem0)}
    pltpu.sync_copy(table_hbm, tab_v.at[pl.ds(0, 4)])
    pltpu.sync_copy(icpt_hbm, icpt_s)
    pltpu.sync_copy(energies_hbm.at[pl.ds(row0, R)], en_v)

    icpt = icpt_s[...]
    lane = lax.iota(jnp.int32, 16)
    hi_mask = lane >= 8
    tvec = tab_v[pl.ds(0, 16)]
    t0, t1, t2, t3 = tvec[0], tvec[1], tvec[2], tvec[3]
    k1 = t1 - t0
    k2 = t2 + t0 - 2.0 * t1
    k3 = t3 - t2 - t1 + t0
    base = t0 * jnp.float32(A)
    zero_i = jnp.zeros((16,), jnp.int32)

    def row_body(sp, r, _):
        sS0 = sS1 = s10 = s11 = s30 = s31 = zero_i
        for j in range(NFULL):
            s = sp[r, pl.ds(j * 16, 16)]
            t = s >> 1
            u = s & t
            if j % 2 == 0:
                sS0 = sS0 + s
                s10 = s10 + t
                s30 = s30 + u
            else:
                sS1 = sS1 + s
                s11 = s11 + t
                s31 = s31 + u
        s = sp[r, pl.ds(NFULL * 16 - 8, 16)]
        s = jnp.where(hi_mask, s, 0)
        t = s >> 1
        u = s & t
        psS[pl.ds(r * SK, 16)] = sS0 + sS1 + s
        ps1[pl.ds(r * SK, 16)] = s10 + s11 + t
        ps3[pl.ds(r * SK, 16)] = s30 + s31 + u
        return _

    def group_body(out_off, g, _):
        col0 = (g * 16 + lane) * SK
        accS = acc1 = acc3 = zero_i
        for j in range(0, 16, 2):
            accS = accS + plsc.load_gather(psS, [col0 + j])
            acc1 = acc1 + plsc.load_gather(ps1, [col0 + j])
            acc3 = acc3 + plsc.load_gather(ps3, [col0 + j])
            accS = accS + plsc.load_gather(psS, [col0 + (j + 1)])
            acc1 = acc1 + plsc.load_gather(ps1, [col0 + (j + 1)])
            acc3 = acc3 + plsc.load_gather(ps3, [col0 + (j + 1)])
        sae = (base
               + k1 * accS.astype(jnp.float32)
               + k2 * acc1.astype(jnp.float32)
               + k3 * acc3.astype(jnp.float32))
        sl = pl.ds(out_off + g * 16, 16)
        out_v[sl] = sae + en_v[sl] + icpt
        return _

    for c in range(NCH):
        if c + 1 < NCH:
            copies[c + 1] = pltpu.async_copy(
                species_hbm.at[pl.ds(row0 + (c + 1) * CR, CR)],
                bufs[(c + 1) % 2], sems[(c + 1) % 2])
        copies[c].wait()
        sp = bufs[c % 2]
        lax.fori_loop(0, CR, functools.partial(row_body, sp), None)
        lax.fori_loop(0, CR // 16,
                      functools.partial(group_body, c * CR), None)

    pltpu.sync_copy(out_v, out_hbm.at[pl.ds(row0, R)])


def kernel(species, energies, self_energies, intercept):
    sae = _sc_energy_adder(
        species,
        energies,
        self_energies,
        jnp.broadcast_to(intercept, (16,)),
    )
    return (species, sae)
